# baseline (speedup 1.0000x reference)
import jax
import jax.numpy as jnp
from jax.experimental import pallas as pl
from jax.experimental.pallas import tpu as pltpu

B, MAXLEN, LOC_LEN, EMB = 16, 128, 512, 32
SU, SL, TU, TL = 100.0, 0.0, 1000.0, 0.0

TT = 32


def _body(traj_len_ref, mat2_ref, vec_ref, tabs_ref, out_ref):
    b = pl.program_id(0)
    tc = pl.program_id(1)
    t0 = tc * TT
    tl_b = traj_len_ref[b]
    t_iota = jax.lax.broadcasted_iota(jnp.int32, (TT, 1), 0) + t0
    m = tl_b > t_iota

    esl = jnp.where(m, tabs_ref[0, 1], tabs_ref[0, 0])
    esu = jnp.where(m, tabs_ref[1, 1], tabs_ref[1, 0])
    etl = jnp.where(m, tabs_ref[2, 1], tabs_ref[2, 0])
    etu = jnp.where(m, tabs_ref[3, 1], tabs_ref[3, 0])

    dt = vec_ref[0]
    A = esl + etl + (etu - etl) * (dt * (1.0 / (TU - TL)))
    C = (esu - esl) * (1.0 / (SU - SL))

    ds = mat2_ref[0]
    out_ref[0] = A[:, None, :] + C[:, None, :] * ds[:, :, None]


def kernel(traj_loc, mat2, vec, traj_len, emb_su, emb_sl, emb_tu, emb_tl):
    tabs = jnp.stack([emb_sl, emb_su, emb_tl, emb_tu])
    grid = (B, MAXLEN // TT)
    out = pl.pallas_call(
        _body,
        grid_spec=pltpu.PrefetchScalarGridSpec(
            num_scalar_prefetch=1,
            grid=grid,
            in_specs=[
                pl.BlockSpec((1, TT, LOC_LEN), lambda b, t, tl: (b, t, 0)),
                pl.BlockSpec((1, TT, 1), lambda b, t, tl: (b, t, 0)),
                pl.BlockSpec((4, 2, EMB), lambda b, t, tl: (0, 0, 0)),
            ],
            out_specs=pl.BlockSpec(
                (1, TT, LOC_LEN, EMB), lambda b, t, tl: (b, t, 0, 0)
            ),
        ),
        out_shape=jax.ShapeDtypeStruct((B, MAXLEN, LOC_LEN, EMB), jnp.float32),
    )(traj_len.astype(jnp.int32), mat2, vec[..., None], tabs)
    return out


# speedup vs baseline: 1.0710x; 1.0710x over previous
import jax
import jax.numpy as jnp
from jax.experimental import pallas as pl
from jax.experimental.pallas import tpu as pltpu

B, MAXLEN, LOC_LEN, EMB = 16, 128, 512, 32
SU, SL, TU, TL = 100.0, 0.0, 1000.0, 0.0

TT = 32


def _body(traj_len_ref, ds_ref, vec_ref, tabs_ref, out_ref):
    tc = pl.program_id(1)
    t0 = tc * TT
    tl_b = traj_len_ref[pl.program_id(0)]
    t_iota = jax.lax.broadcasted_iota(jnp.int32, (TT, 1), 0) + t0
    m = tl_b > t_iota

    esl = jnp.where(m, tabs_ref[0, 1], tabs_ref[0, 0])
    esu = jnp.where(m, tabs_ref[1, 1], tabs_ref[1, 0])
    etl = jnp.where(m, tabs_ref[2, 1], tabs_ref[2, 0])
    etu = jnp.where(m, tabs_ref[3, 1], tabs_ref[3, 0])

    dt = vec_ref[0]
    A = esl + etl + (etu - etl) * (dt * (1.0 / (TU - TL)))
    C = (esu - esl) * (1.0 / (SU - SL))

    ds4 = ds_ref[0]
    ds_exp = jnp.concatenate(
        [jnp.broadcast_to(ds4[:, :, d : d + 1], (TT, 128, 32)) for d in range(4)],
        axis=-1,
    )
    out_ref[0] = A[:, None, :] + C[:, None, :] * ds_exp


def kernel(traj_loc, mat2, vec, traj_len, emb_su, emb_sl, emb_tu, emb_tl):
    tabs = jnp.stack([emb_sl, emb_su, emb_tl, emb_tu])
    tabs = jnp.tile(tabs, (1, 1, 128 // EMB))
    ds4 = mat2.reshape(B, MAXLEN, LOC_LEN // 4, 4)
    grid = (B, MAXLEN // TT)
    out = pl.pallas_call(
        _body,
        grid_spec=pltpu.PrefetchScalarGridSpec(
            num_scalar_prefetch=1,
            grid=grid,
            in_specs=[
                pl.BlockSpec((1, TT, LOC_LEN // 4, 4), lambda b, t, tl: (b, t, 0, 0)),
                pl.BlockSpec((1, TT, 1), lambda b, t, tl: (b, t, 0)),
                pl.BlockSpec((4, 2, 128), lambda b, t, tl: (0, 0, 0)),
            ],
            out_specs=pl.BlockSpec(
                (1, TT, LOC_LEN // 4, 4 * EMB), lambda b, t, tl: (b, t, 0, 0)
            ),
        ),
        out_shape=jax.ShapeDtypeStruct(
            (B, MAXLEN, LOC_LEN // 4, 4 * EMB), jnp.float32
        ),
    )(traj_len.astype(jnp.int32), ds4, vec[..., None], tabs)
    return out.reshape(B, MAXLEN, LOC_LEN, EMB)


# speedup vs baseline: 6.3850x; 5.9617x over previous
import jax
import jax.numpy as jnp
from jax.experimental import pallas as pl
from jax.experimental.pallas import tpu as pltpu

B, MAXLEN, LOC_LEN, EMB = 16, 128, 512, 32
SU, SL, TU, TL = 100.0, 0.0, 1000.0, 0.0

TT = 32


def _body(traj_len_ref, ds_ref, vec_ref, tabs_ref, out_ref):
    t0 = pl.program_id(1) * TT
    tl_b = traj_len_ref[pl.program_id(0)]
    t_iota = jax.lax.broadcasted_iota(jnp.int32, (TT, 1, 1), 0) + t0
    m = tl_b > t_iota

    p0 = tabs_ref[0, 0] + tabs_ref[2, 0]
    p1 = tabs_ref[0, 1] + tabs_ref[2, 1]
    q0 = (tabs_ref[3, 0] - tabs_ref[2, 0]) * (1.0 / (TU - TL))
    q1 = (tabs_ref[3, 1] - tabs_ref[2, 1]) * (1.0 / (TU - TL))
    r0 = (tabs_ref[1, 0] - tabs_ref[0, 0]) * (1.0 / (SU - SL))
    r1 = (tabs_ref[1, 1] - tabs_ref[0, 1]) * (1.0 / (SU - SL))

    p = jnp.where(m, p1, p0)
    q = jnp.where(m, q1, q0)
    r = jnp.where(m, r1, r0)

    dt = vec_ref[0]
    s = p + q * dt
    ds = ds_ref[0]
    out_ref[0] = s + r * ds


def kernel(traj_loc, mat2, vec, traj_len, emb_su, emb_sl, emb_tu, emb_tl):
    tabs = jnp.stack([emb_sl, emb_su, emb_tl, emb_tu])[..., None]
    grid = (B, MAXLEN // TT)
    out = pl.pallas_call(
        _body,
        grid_spec=pltpu.PrefetchScalarGridSpec(
            num_scalar_prefetch=1,
            grid=grid,
            in_specs=[
                pl.BlockSpec((1, TT, 1, LOC_LEN), lambda b, t, tl: (b, t, 0, 0)),
                pl.BlockSpec((1, TT, 1, 1), lambda b, t, tl: (b, t, 0, 0)),
                pl.BlockSpec((4, 2, EMB, 1), lambda b, t, tl: (0, 0, 0, 0)),
            ],
            out_specs=pl.BlockSpec(
                (1, TT, EMB, LOC_LEN), lambda b, t, tl: (b, t, 0, 0)
            ),
        ),
        out_shape=jax.ShapeDtypeStruct((B, MAXLEN, EMB, LOC_LEN), jnp.float32),
    )(
        traj_len.astype(jnp.int32),
        mat2[:, :, None, :],
        vec[:, :, None, None],
        tabs,
    )
    return jnp.swapaxes(out, 2, 3)


# speedup vs baseline: 8.8466x; 1.3855x over previous
import jax
import jax.numpy as jnp
from jax.experimental import pallas as pl
from jax.experimental.pallas import tpu as pltpu

B, MAXLEN, LOC_LEN, EMB = 16, 128, 512, 32
SU, SL, TU, TL = 100.0, 0.0, 1000.0, 0.0

TT = 128


def _body(traj_len_ref, ds_ref, vec_ref, tabs_ref, out_ref):
    tl_b = traj_len_ref[pl.program_id(0)]
    t_iota = jax.lax.broadcasted_iota(jnp.int32, (TT, 1, 1), 0)
    m = tl_b > t_iota

    p0 = tabs_ref[0, 0] + tabs_ref[2, 0]
    p1 = tabs_ref[0, 1] + tabs_ref[2, 1]
    q0 = (tabs_ref[3, 0] - tabs_ref[2, 0]) * (1.0 / (TU - TL))
    q1 = (tabs_ref[3, 1] - tabs_ref[2, 1]) * (1.0 / (TU - TL))
    r0 = (tabs_ref[1, 0] - tabs_ref[0, 0]) * (1.0 / (SU - SL))
    r1 = (tabs_ref[1, 1] - tabs_ref[0, 1]) * (1.0 / (SU - SL))

    p = jnp.where(m, p1, p0)
    q = jnp.where(m, q1, q0)
    r = jnp.where(m, r1, r0)

    dt = vec_ref[0]
    s = p + q * dt
    ds = ds_ref[0]
    out_ref[0] = s + r * ds


def kernel(traj_loc, mat2, vec, traj_len, emb_su, emb_sl, emb_tu, emb_tl):
    tabs = jnp.stack([emb_sl, emb_su, emb_tl, emb_tu])[..., None]
    grid = (B,)
    out = pl.pallas_call(
        _body,
        grid_spec=pltpu.PrefetchScalarGridSpec(
            num_scalar_prefetch=1,
            grid=grid,
            in_specs=[
                pl.BlockSpec((1, TT, 1, LOC_LEN), lambda b, tl: (b, 0, 0, 0)),
                pl.BlockSpec((1, TT, 1, 1), lambda b, tl: (b, 0, 0, 0)),
                pl.BlockSpec((4, 2, EMB, 1), lambda b, tl: (0, 0, 0, 0)),
            ],
            out_specs=pl.BlockSpec(
                (1, TT, EMB, LOC_LEN), lambda b, tl: (b, 0, 0, 0)
            ),
        ),
        out_shape=jax.ShapeDtypeStruct((B, MAXLEN, EMB, LOC_LEN), jnp.float32),
    )(
        traj_len.astype(jnp.int32),
        mat2[:, :, None, :],
        vec[:, :, None, None],
        tabs,
    )
    return jnp.swapaxes(out, 2, 3)
